# baseline (speedup 1.0000x reference)
import functools

import jax
import jax.numpy as jnp
from jax import lax
from jax.experimental import pallas as pl
from jax.experimental.pallas import tpu as pltpu
from jax.experimental.pallas import tpu_sc as plsc

NC = 2
NS = 16
NW = NC * NS
L = 16

B = 128
CHUNK = 8

NUM_GRAPHS = 64


def _ceil_to(a, m):
  return (a + m - 1) // m * m


def _sc_setup_body(n_nodes, n_rel, ng, nbins, dst_hbm, et_hbm, src_hbm,
                   w_hbm, gidx_hbm, cnt_sh, zbuf, cb0, cb1, etb, comp_all,
                   w_all, gidx_all, ones_v, sem_a, sem_b):
  s_id = lax.axis_index("s")
  c_id = lax.axis_index("c")
  wid = s_id * NC + c_id
  ng2 = 2 * ng

  bins_per_tile = nbins // NS
  def _z(i, _):
    zbuf[pl.ds(i * L, L)] = jnp.zeros((L,), jnp.float32)
    return 0
  lax.fori_loop(0, bins_per_tile // L, _z, 0)
  pltpu.sync_copy(zbuf, cnt_sh.at[pl.ds(s_id * bins_per_tile, bins_per_tile)])
  def _o(i, _):
    ones_v[pl.ds(i * L, L)] = jnp.ones((L,), jnp.float32)
    return 0
  lax.fori_loop(0, B // L, _o, 0)
  plsc.subcore_barrier()

  pltpu.sync_copy(dst_hbm.at[pl.ds(s_id * ng2, ng2)], cb0)
  pltpu.sync_copy(et_hbm.at[pl.ds(s_id * ng2, ng2)], cb1)
  def _comp(i, _):
    j = i // (B // L)
    k = i % (B // L)
    dv = cb0[j, pl.ds(k * L, L)]
    ev = cb1[j, pl.ds(k * L, L)]
    comp_all[j, pl.ds(k * L, L)] = dv * n_rel + ev
    return 0
  lax.fori_loop(0, ng2 * (B // L), _comp, 0)
  def _fire(g, _):
    pltpu.async_copy(ones_v, cnt_sh.at[comp_all.at[g]], sem_a, add=True)
    return 0
  lax.fori_loop(0, ng2, _fire, 0)
  def _drain(g, _):
    pltpu.make_async_copy(ones_v, cnt_sh.at[comp_all.at[g]], sem_a).wait()
    return 0
  lax.fori_loop(0, ng2, _drain, 0)
  plsc.subcore_barrier()

  pltpu.sync_copy(cnt_sh.at[pl.ds(s_id * bins_per_tile, bins_per_tile)], zbuf)
  def _r(i, _):
    v = zbuf[pl.ds(i * L, L)]
    zbuf[pl.ds(i * L, L)] = 1.0 / jnp.maximum(v, 1.0)
    return 0
  lax.fori_loop(0, bins_per_tile // L, _r, 0)
  pltpu.sync_copy(zbuf, cnt_sh.at[pl.ds(s_id * bins_per_tile, bins_per_tile)])
  plsc.subcore_barrier()

  pltpu.sync_copy(src_hbm.at[pl.ds(wid * ng, ng)], cb0.at[pl.ds(0, ng)])
  pltpu.sync_copy(dst_hbm.at[pl.ds(wid * ng, ng)], cb1.at[pl.ds(0, ng)])
  pltpu.sync_copy(et_hbm.at[pl.ds(wid * ng, ng)], etb)
  def _gix(i, _):
    j = i // (B // L)
    k = i % (B // L)
    sv = cb0[j, pl.ds(k * L, L)]
    dv = cb1[j, pl.ds(k * L, L)]
    ev = etb[j, pl.ds(k * L, L)]
    comp_all[j, pl.ds(k * L, L)] = dv * n_rel + ev
    gidx_all[j, pl.ds(k * L, L)] = ev * n_nodes + sv
    return 0
  lax.fori_loop(0, ng * (B // L), _gix, 0)
  def _wfire(g, _):
    pltpu.async_copy(cnt_sh.at[comp_all.at[g]], w_all.at[g], sem_b)
    return 0
  lax.fori_loop(0, ng, _wfire, 0)
  def _wdrain(g, _):
    pltpu.make_async_copy(cnt_sh.at[comp_all.at[g]], w_all.at[g], sem_b).wait()
    return 0
  lax.fori_loop(0, ng, _wdrain, 0)
  pltpu.sync_copy(w_all, w_hbm.at[pl.ds(wid * ng, ng)])
  pltpu.sync_copy(gidx_all, gidx_hbm.at[pl.ds(wid * ng, ng)])


def _make_sc_setup(n_nodes, n_rel, ng, nbins):
  mesh = plsc.VectorSubcoreMesh(core_axis_name="c", subcore_axis_name="s")
  body = functools.partial(_sc_setup_body, n_nodes, n_rel, ng, nbins)
  return pl.kernel(
      body,
      out_type=(
          jax.ShapeDtypeStruct((NW * ng, B), jnp.float32),
          jax.ShapeDtypeStruct((NW * ng, B), jnp.int32),
      ),
      mesh=mesh,
      scratch_types=[
          pltpu.VMEM_SHARED((nbins,), jnp.float32),
          pltpu.VMEM((nbins // NS,), jnp.float32),
          pltpu.VMEM((2 * ng, B), jnp.int32),
          pltpu.VMEM((2 * ng, B), jnp.int32),
          pltpu.VMEM((ng, B), jnp.int32),
          pltpu.VMEM((2 * ng, B), jnp.int32),
          pltpu.VMEM((ng, B), jnp.float32),
          pltpu.VMEM((ng, B), jnp.int32),
          pltpu.VMEM((B,), jnp.float32),
          pltpu.SemaphoreType.DMA,
          pltpu.SemaphoreType.DMA,
      ],
      compiler_params=pltpu.CompilerParams(needs_layout_passes=False),
      name="rgcn_sc_setup",
  )


def _sc_layer_body(n_acc, d, ng, xr_hbm, gidx_hbm, dst_hbm, w_hbm,
                   out_hbm, acc_sh, r0, r1, g0, g1, d0, d1, w0, w1,
                   sg0, sg1, ss0, ss1, se0, se1):
  s_id = lax.axis_index("s")
  c_id = lax.axis_index("c")
  wid = s_id * NC + c_id
  rows = (r0, r1)
  gts = (g0, g1)
  dts = (d0, d1)
  wts = (w0, w1)
  sgs = (sg0, sg1)
  sss = (ss0, ss1)
  ses = (se0, se1)
  nchunks = ng // CHUNK

  rows_per_tile = n_acc // NS
  def _z(i, _):
    for c8 in range(d // L):
      r0[i, pl.ds(c8 * L, L)] = jnp.zeros((L,), jnp.float32)
    return 0
  lax.fori_loop(0, B, _z, 0)
  for k in range(rows_per_tile // B):
    pltpu.sync_copy(r0, acc_sh.at[pl.ds(s_id * rows_per_tile + k * B, B)])

  def _ech_start(cc, p):
    off = pl.multiple_of(wid * ng + cc * CHUNK, 8)
    pltpu.async_copy(gidx_hbm.at[pl.ds(off, CHUNK)], gts[p], ses[p])
    pltpu.async_copy(dst_hbm.at[pl.ds(off, CHUNK)], dts[p], ses[p])
    pltpu.async_copy(w_hbm.at[pl.ds(off, CHUNK)], wts[p], ses[p])

  def _ech_wait(p):
    base = pl.multiple_of(wid * ng, 8)
    pltpu.make_async_copy(gidx_hbm.at[pl.ds(base, CHUNK)], gts[p],
                          ses[p]).wait()
    pltpu.make_async_copy(dst_hbm.at[pl.ds(base, CHUNK)], dts[p],
                          ses[p]).wait()
    pltpu.make_async_copy(w_hbm.at[pl.ds(base, CHUNK)], wts[p],
                          ses[p]).wait()

  _ech_start(0, 0)
  plsc.subcore_barrier()

  def _scale(buf, wt, gg):
    def _body(j, _):
      wv = wt[gg, pl.ds(j * L, L)]
      for k in range(L):
        w = wv[k]
        i = j * L + k
        for c8 in range(d // L):
          buf[i, pl.ds(c8 * L, L)] = buf[i, pl.ds(c8 * L, L)] * w
      return 0
    lax.fori_loop(0, B // L, _body, 0)

  def _gwait(b):
    pltpu.make_async_copy(xr_hbm.at[gts[0].at[0]], rows[b], sgs[b]).wait()

  def _swait(b):
    pltpu.make_async_copy(rows[b], acc_sh.at[dts[0].at[0]], sss[b]).wait()

  def _chunk(cc, p):
    _ech_wait(p)
    @pl.when(cc + 1 < nchunks)
    def _pref():
      _ech_start(cc + 1, 1 - p)
    for pair in range(CHUNK // 2):
      gg0 = 2 * pair
      gg1 = 2 * pair + 1
      glob0 = cc * CHUNK + gg0
      @pl.when(glob0 >= 2)
      def _w0():
        _swait(0)
        _swait(1)
      pltpu.async_copy(xr_hbm.at[gts[p].at[gg0]], rows[0], sgs[0])
      pltpu.async_copy(xr_hbm.at[gts[p].at[gg1]], rows[1], sgs[1])
      _gwait(0)
      _scale(r0, wts[p], gg0)
      pltpu.async_copy(r0, acc_sh.at[dts[p].at[gg0]], sss[0], add=True)
      _gwait(1)
      _scale(r1, wts[p], gg1)
      pltpu.async_copy(r1, acc_sh.at[dts[p].at[gg1]], sss[1], add=True)

  def _iter(t, _):
    _chunk(2 * t, 0)
    _chunk(2 * t + 1, 1)
    return 0
  lax.fori_loop(0, nchunks // 2, _iter, 0)
  _swait(0)
  _swait(1)
  plsc.subcore_barrier()

  base = pl.multiple_of(c_id * n_acc + s_id * rows_per_tile, 8)
  pltpu.sync_copy(acc_sh.at[pl.ds(s_id * rows_per_tile, rows_per_tile)],
                  out_hbm.at[pl.ds(base, rows_per_tile)])


def _make_sc_layer(n_acc, d, ng):
  mesh = plsc.VectorSubcoreMesh(core_axis_name="c", subcore_axis_name="s")
  body = functools.partial(_sc_layer_body, n_acc, d, ng)
  return pl.kernel(
      body,
      out_type=jax.ShapeDtypeStruct((NC * n_acc, d), jnp.float32),
      mesh=mesh,
      scratch_types=[
          pltpu.VMEM_SHARED((n_acc, d), jnp.float32),
          pltpu.VMEM((B, d), jnp.float32),
          pltpu.VMEM((B, d), jnp.float32),
          pltpu.VMEM((CHUNK, B), jnp.int32),
          pltpu.VMEM((CHUNK, B), jnp.int32),
          pltpu.VMEM((CHUNK, B), jnp.int32),
          pltpu.VMEM((CHUNK, B), jnp.int32),
          pltpu.VMEM((CHUNK, B), jnp.float32),
          pltpu.VMEM((CHUNK, B), jnp.float32),
          pltpu.SemaphoreType.DMA,
          pltpu.SemaphoreType.DMA,
          pltpu.SemaphoreType.DMA,
          pltpu.SemaphoreType.DMA,
          pltpu.SemaphoreType.DMA,
          pltpu.SemaphoreType.DMA,
      ],
      name="rgcn_sc_layer",
  )


def _mm_body(x_ref, w_ref, o_ref):
  o_ref[0] = jnp.dot(x_ref[...], w_ref[0],
                     preferred_element_type=jnp.float32)


def _relation_matmul(x, w_stack, n_blk):
  n, d = x.shape
  r1 = w_stack.shape[0]
  grid = (r1, n // n_blk)
  return pl.pallas_call(
      _mm_body,
      grid=grid,
      in_specs=[
          pl.BlockSpec((n_blk, d), lambda r, i: (i, 0)),
          pl.BlockSpec((1, d, d), lambda r, i: (r, 0, 0)),
      ],
      out_specs=pl.BlockSpec((1, n_blk, d), lambda r, i: (r, i, 0)),
      out_shape=jax.ShapeDtypeStruct((r1, n, d), jnp.float32),
  )(x, w_stack)


def _combine_body(a0_ref, a1_ref, xr_ref, b_ref, o_ref):
  o_ref[...] = jnp.maximum(
      a0_ref[...] + a1_ref[...] + xr_ref[...] + b_ref[...], 0.0)


def _combine(acc0, acc1, xr_root, b, n_blk):
  n, d = acc0.shape
  grid = (n // n_blk,)
  return pl.pallas_call(
      _combine_body,
      grid=grid,
      in_specs=[
          pl.BlockSpec((n_blk, d), lambda i: (i, 0)),
          pl.BlockSpec((n_blk, d), lambda i: (i, 0)),
          pl.BlockSpec((n_blk, d), lambda i: (i, 0)),
          pl.BlockSpec((1, d), lambda i: (0, 0)),
      ],
      out_specs=pl.BlockSpec((n_blk, d), lambda i: (i, 0)),
      out_shape=jax.ShapeDtypeStruct((n, d), jnp.float32),
  )(acc0, acc1, xr_root, b.reshape(1, d))


def _pool_body(n_groups, h_ref, batch_ref, wc_ref, bc_ref, o_ref):
  npad = h_ref.shape[0]
  ids = lax.broadcasted_iota(jnp.int32, (n_groups, npad), 0)
  onehot = jnp.where(ids == batch_ref[...], 1.0, 0.0)
  sums = jnp.dot(onehot, h_ref[...], preferred_element_type=jnp.float32)
  cnt = jnp.sum(onehot, axis=1, keepdims=True)
  g = sums / jnp.maximum(cnt, 1.0)
  o_ref[...] = jnp.dot(g, wc_ref[...],
                       preferred_element_type=jnp.float32) + bc_ref[...]


def _pool_classify(h_pad, batch_pad, wc, bc, n_groups):
  npad, d = h_pad.shape
  c = wc.shape[1]
  return pl.pallas_call(
      functools.partial(_pool_body, n_groups),
      in_specs=[
          pl.BlockSpec((npad, d), lambda: (0, 0)),
          pl.BlockSpec((n_groups, npad), lambda: (0, 0)),
          pl.BlockSpec((d, c), lambda: (0, 0)),
          pl.BlockSpec((1, c), lambda: (0, 0)),
      ],
      out_specs=pl.BlockSpec((n_groups, c), lambda: (0, 0)),
      out_shape=jax.ShapeDtypeStruct((n_groups, c), jnp.float32),
  )(h_pad, jnp.broadcast_to(batch_pad[None, :], (n_groups, npad)), wc,
    bc.reshape(1, c))


def kernel(x, edge_index, edge_type, batch, W1, root1, b1, W2, root2, b2,
           Wc, bc):
  n, d = x.shape
  r = W1.shape[0]
  e = edge_index.shape[1]
  n_groups = NUM_GRAPHS

  src = edge_index[0].astype(jnp.int32)
  dst = edge_index[1].astype(jnp.int32)
  et = edge_type.astype(jnp.int32)
  batch32 = batch.astype(jnp.int32)

  ng = _ceil_to(-(-e // (NW * B)), 2 * CHUNK)
  e_pad = NW * B * ng
  n_acc = _ceil_to(n + 1, NS * B)
  nbins = n_acc * r
  pad = e_pad - e

  src_p = jnp.concatenate([src, jnp.zeros((pad,), jnp.int32)]).reshape(-1, B)
  dst_p = jnp.concatenate([dst, jnp.full((pad,), n, jnp.int32)]).reshape(-1, B)
  et_p = jnp.concatenate([et, jnp.zeros((pad,), jnp.int32)]).reshape(-1, B)

  w_edge, gidx = _make_sc_setup(n, r, ng, nbins)(dst_p, et_p, src_p)

  sc_layer = _make_sc_layer(n_acc, d, ng)
  n_blk = 1000

  def layer(h, w_rel, root, b):
    w_stack = jnp.concatenate([w_rel, root[None]], axis=0)
    xr = _relation_matmul(h, w_stack, n_blk)
    xr_flat = xr[:r].reshape(r * n, d)
    acc = sc_layer(xr_flat, gidx, dst_p, w_edge)
    acc0 = acc[:n]
    acc1 = acc[n_acc:n_acc + n]
    return _combine(acc0, acc1, xr[r], b, n_blk)

  h = layer(x, W1, root1, b1)
  h = layer(h, W2, root2, b2)

  n_pad = _ceil_to(n, B)
  h_pad = jnp.pad(h, ((0, n_pad - n), (0, 0)))
  batch_pad = jnp.concatenate(
      [batch32, jnp.full((n_pad - n,), n_groups, jnp.int32)])
  return _pool_classify(h_pad, batch_pad, Wc, bc, n_groups)


# speedup vs baseline: 1.0002x; 1.0002x over previous
import functools

import jax
import jax.numpy as jnp
from jax import lax
from jax.experimental import pallas as pl
from jax.experimental.pallas import tpu as pltpu
from jax.experimental.pallas import tpu_sc as plsc

NC = 2
NS = 16
NW = NC * NS
L = 16

B = 128
CHUNK = 8

NUM_GRAPHS = 64


def _ceil_to(a, m):
  return (a + m - 1) // m * m


def _sc_setup_body(n_nodes, n_rel, ng, nbins, dst_hbm, et_hbm, src_hbm,
                   w_hbm, gidx_hbm, cnt_sh, zbuf, cb0, cb1, etb, comp_all,
                   w_all, gidx_all, ones_v, sem_a, sem_b):
  s_id = lax.axis_index("s")
  c_id = lax.axis_index("c")
  wid = s_id * NC + c_id
  ng2 = 2 * ng

  bins_per_tile = nbins // NS
  def _z(i, _):
    zbuf[pl.ds(i * L, L)] = jnp.zeros((L,), jnp.float32)
    return 0
  lax.fori_loop(0, bins_per_tile // L, _z, 0)
  pltpu.sync_copy(zbuf, cnt_sh.at[pl.ds(s_id * bins_per_tile, bins_per_tile)])
  def _o(i, _):
    ones_v[pl.ds(i * L, L)] = jnp.ones((L,), jnp.float32)
    return 0
  lax.fori_loop(0, B // L, _o, 0)
  plsc.subcore_barrier()

  pltpu.sync_copy(dst_hbm.at[pl.ds(s_id * ng2, ng2)], cb0)
  pltpu.sync_copy(et_hbm.at[pl.ds(s_id * ng2, ng2)], cb1)
  def _comp(i, _):
    j = i // (B // L)
    k = i % (B // L)
    dv = cb0[j, pl.ds(k * L, L)]
    ev = cb1[j, pl.ds(k * L, L)]
    comp_all[j, pl.ds(k * L, L)] = dv * n_rel + ev
    return 0
  lax.fori_loop(0, ng2 * (B // L), _comp, 0)
  def _fire(g, _):
    pltpu.async_copy(ones_v, cnt_sh.at[comp_all.at[g]], sem_a, add=True)
    return 0
  lax.fori_loop(0, ng2, _fire, 0)
  def _drain(g, _):
    pltpu.make_async_copy(ones_v, cnt_sh.at[comp_all.at[g]], sem_a).wait()
    return 0
  lax.fori_loop(0, ng2, _drain, 0)
  plsc.subcore_barrier()

  pltpu.sync_copy(cnt_sh.at[pl.ds(s_id * bins_per_tile, bins_per_tile)], zbuf)
  def _r(i, _):
    v = zbuf[pl.ds(i * L, L)]
    zbuf[pl.ds(i * L, L)] = 1.0 / jnp.maximum(v, 1.0)
    return 0
  lax.fori_loop(0, bins_per_tile // L, _r, 0)
  pltpu.sync_copy(zbuf, cnt_sh.at[pl.ds(s_id * bins_per_tile, bins_per_tile)])
  plsc.subcore_barrier()

  pltpu.sync_copy(src_hbm.at[pl.ds(wid * ng, ng)], cb0.at[pl.ds(0, ng)])
  pltpu.sync_copy(dst_hbm.at[pl.ds(wid * ng, ng)], cb1.at[pl.ds(0, ng)])
  pltpu.sync_copy(et_hbm.at[pl.ds(wid * ng, ng)], etb)
  def _gix(i, _):
    j = i // (B // L)
    k = i % (B // L)
    sv = cb0[j, pl.ds(k * L, L)]
    dv = cb1[j, pl.ds(k * L, L)]
    ev = etb[j, pl.ds(k * L, L)]
    comp_all[j, pl.ds(k * L, L)] = dv * n_rel + ev
    gidx_all[j, pl.ds(k * L, L)] = ev * n_nodes + sv
    return 0
  lax.fori_loop(0, ng * (B // L), _gix, 0)
  def _wfire(g, _):
    pltpu.async_copy(cnt_sh.at[comp_all.at[g]], w_all.at[g], sem_b)
    return 0
  lax.fori_loop(0, ng, _wfire, 0)
  def _wdrain(g, _):
    pltpu.make_async_copy(cnt_sh.at[comp_all.at[g]], w_all.at[g], sem_b).wait()
    return 0
  lax.fori_loop(0, ng, _wdrain, 0)
  pltpu.sync_copy(w_all, w_hbm.at[pl.ds(wid * ng, ng)])
  pltpu.sync_copy(gidx_all, gidx_hbm.at[pl.ds(wid * ng, ng)])


def _make_sc_setup(n_nodes, n_rel, ng, nbins):
  mesh = plsc.VectorSubcoreMesh(core_axis_name="c", subcore_axis_name="s")
  body = functools.partial(_sc_setup_body, n_nodes, n_rel, ng, nbins)
  return pl.kernel(
      body,
      out_type=(
          jax.ShapeDtypeStruct((NW * ng, B), jnp.float32),
          jax.ShapeDtypeStruct((NW * ng, B), jnp.int32),
      ),
      mesh=mesh,
      scratch_types=[
          pltpu.VMEM_SHARED((nbins,), jnp.float32),
          pltpu.VMEM((nbins // NS,), jnp.float32),
          pltpu.VMEM((2 * ng, B), jnp.int32),
          pltpu.VMEM((2 * ng, B), jnp.int32),
          pltpu.VMEM((ng, B), jnp.int32),
          pltpu.VMEM((2 * ng, B), jnp.int32),
          pltpu.VMEM((ng, B), jnp.float32),
          pltpu.VMEM((ng, B), jnp.int32),
          pltpu.VMEM((B,), jnp.float32),
          pltpu.SemaphoreType.DMA,
          pltpu.SemaphoreType.DMA,
      ],
      compiler_params=pltpu.CompilerParams(needs_layout_passes=False),
      name="rgcn_sc_setup",
  )


def _sc_layer_body(n_acc, d, ng, xr_hbm, gidx_hbm, dst_hbm, w_hbm,
                   out_hbm, acc_sh, r0, r1, g0, g1, d0, d1, w0, w1,
                   sg0, sg1, ss0, ss1, se0, se1):
  s_id = lax.axis_index("s")
  c_id = lax.axis_index("c")
  wid = s_id * NC + c_id
  rows = (r0, r1)
  gts = (g0, g1)
  dts = (d0, d1)
  wts = (w0, w1)
  sgs = (sg0, sg1)
  sss = (ss0, ss1)
  ses = (se0, se1)
  nchunks = ng // CHUNK

  rows_per_tile = n_acc // NS
  def _z(i, _):
    for c8 in range(d // L):
      r0[i, pl.ds(c8 * L, L)] = jnp.zeros((L,), jnp.float32)
    return 0
  lax.fori_loop(0, B, _z, 0)
  for k in range(rows_per_tile // B):
    pltpu.sync_copy(r0, acc_sh.at[pl.ds(s_id * rows_per_tile + k * B, B)])

  def _ech_start(cc, p):
    off = pl.multiple_of(wid * ng + cc * CHUNK, 8)
    pltpu.async_copy(gidx_hbm.at[pl.ds(off, CHUNK)], gts[p], ses[p])
    pltpu.async_copy(dst_hbm.at[pl.ds(off, CHUNK)], dts[p], ses[p])
    pltpu.async_copy(w_hbm.at[pl.ds(off, CHUNK)], wts[p], ses[p])

  def _ech_wait(p):
    base = pl.multiple_of(wid * ng, 8)
    pltpu.make_async_copy(gidx_hbm.at[pl.ds(base, CHUNK)], gts[p],
                          ses[p]).wait()
    pltpu.make_async_copy(dst_hbm.at[pl.ds(base, CHUNK)], dts[p],
                          ses[p]).wait()
    pltpu.make_async_copy(w_hbm.at[pl.ds(base, CHUNK)], wts[p],
                          ses[p]).wait()

  _ech_start(0, 0)
  plsc.subcore_barrier()

  def _scale(buf, wt, gg):
    def _body(j, _):
      wv = wt[gg, pl.ds(j * L, L)]
      for k in range(L):
        w = wv[k]
        i = j * L + k
        for c8 in range(d // L):
          buf[i, pl.ds(c8 * L, L)] = buf[i, pl.ds(c8 * L, L)] * w
      return 0
    lax.fori_loop(0, B // L, _body, 0)

  def _gwait(b):
    pltpu.make_async_copy(xr_hbm.at[gts[0].at[0]], rows[b], sgs[b]).wait()

  def _swait(b):
    pltpu.make_async_copy(rows[b], acc_sh.at[dts[0].at[0]], sss[b]).wait()

  def _chunk(cc, p):
    _ech_wait(p)
    @pl.when(cc + 1 < nchunks)
    def _pref():
      _ech_start(cc + 1, 1 - p)
    for pair in range(CHUNK // 2):
      gg0 = 2 * pair
      gg1 = 2 * pair + 1
      glob0 = cc * CHUNK + gg0
      @pl.when(glob0 >= 2)
      def _w0():
        _swait(0)
        _swait(1)
      pltpu.async_copy(xr_hbm.at[gts[p].at[gg0]], rows[0], sgs[0])
      pltpu.async_copy(xr_hbm.at[gts[p].at[gg1]], rows[1], sgs[1])
      _gwait(0)
      _scale(r0, wts[p], gg0)
      pltpu.async_copy(r0, acc_sh.at[dts[p].at[gg0]], sss[0], add=True)
      _gwait(1)
      _scale(r1, wts[p], gg1)
      pltpu.async_copy(r1, acc_sh.at[dts[p].at[gg1]], sss[1], add=True)

  def _iter(t, _):
    _chunk(2 * t, 0)
    _chunk(2 * t + 1, 1)
    return 0
  lax.fori_loop(0, nchunks // 2, _iter, 0)
  _swait(0)
  _swait(1)
  plsc.subcore_barrier()

  base = pl.multiple_of(c_id * n_acc + s_id * rows_per_tile, 8)
  pltpu.sync_copy(acc_sh.at[pl.ds(s_id * rows_per_tile, rows_per_tile)],
                  out_hbm.at[pl.ds(base, rows_per_tile)])


def _make_sc_layer(n_acc, d, ng):
  mesh = plsc.VectorSubcoreMesh(core_axis_name="c", subcore_axis_name="s")
  body = functools.partial(_sc_layer_body, n_acc, d, ng)
  return pl.kernel(
      body,
      out_type=jax.ShapeDtypeStruct((NC * n_acc, d), jnp.float32),
      mesh=mesh,
      scratch_types=[
          pltpu.VMEM_SHARED((n_acc, d), jnp.float32),
          pltpu.VMEM((B, d), jnp.float32),
          pltpu.VMEM((B, d), jnp.float32),
          pltpu.VMEM((CHUNK, B), jnp.int32),
          pltpu.VMEM((CHUNK, B), jnp.int32),
          pltpu.VMEM((CHUNK, B), jnp.int32),
          pltpu.VMEM((CHUNK, B), jnp.int32),
          pltpu.VMEM((CHUNK, B), jnp.float32),
          pltpu.VMEM((CHUNK, B), jnp.float32),
          pltpu.SemaphoreType.DMA,
          pltpu.SemaphoreType.DMA,
          pltpu.SemaphoreType.DMA,
          pltpu.SemaphoreType.DMA,
          pltpu.SemaphoreType.DMA,
          pltpu.SemaphoreType.DMA,
      ],
      name="rgcn_sc_layer",
  )


def _mm_body(x_ref, w_ref, o_ref):
  o_ref[0] = jnp.dot(x_ref[...], w_ref[0],
                     preferred_element_type=jnp.float32)


def _relation_matmul(x, w_stack, n_blk):
  n, d = x.shape
  r1 = w_stack.shape[0]
  grid = (r1, n // n_blk)
  return pl.pallas_call(
      _mm_body,
      grid=grid,
      in_specs=[
          pl.BlockSpec((n_blk, d), lambda r, i: (i, 0)),
          pl.BlockSpec((1, d, d), lambda r, i: (r, 0, 0)),
      ],
      out_specs=pl.BlockSpec((1, n_blk, d), lambda r, i: (r, i, 0)),
      out_shape=jax.ShapeDtypeStruct((r1, n, d), jnp.float32),
  )(x, w_stack)


def _combine_body(a0_ref, a1_ref, xr_ref, b_ref, o_ref):
  o_ref[...] = jnp.maximum(
      a0_ref[...] + a1_ref[...] + xr_ref[...] + b_ref[...], 0.0)


def _combine(acc0, acc1, xr_root, b, n_blk):
  n, d = acc0.shape
  grid = (n // n_blk,)
  return pl.pallas_call(
      _combine_body,
      grid=grid,
      in_specs=[
          pl.BlockSpec((n_blk, d), lambda i: (i, 0)),
          pl.BlockSpec((n_blk, d), lambda i: (i, 0)),
          pl.BlockSpec((n_blk, d), lambda i: (i, 0)),
          pl.BlockSpec((1, d), lambda i: (0, 0)),
      ],
      out_specs=pl.BlockSpec((n_blk, d), lambda i: (i, 0)),
      out_shape=jax.ShapeDtypeStruct((n, d), jnp.float32),
  )(acc0, acc1, xr_root, b.reshape(1, d))


def _pool_body(n_groups, h_ref, batch_ref, wc_ref, bc_ref, o_ref):
  npad = h_ref.shape[0]
  ids = lax.broadcasted_iota(jnp.int32, (n_groups, npad), 0)
  onehot = jnp.where(ids == batch_ref[...], 1.0, 0.0)
  sums = jnp.dot(onehot, h_ref[...], preferred_element_type=jnp.float32)
  cnt = jnp.sum(onehot, axis=1, keepdims=True)
  g = sums / jnp.maximum(cnt, 1.0)
  o_ref[...] = jnp.dot(g, wc_ref[...],
                       preferred_element_type=jnp.float32) + bc_ref[...]


def _pool_classify(h_pad, batch_pad, wc, bc, n_groups):
  npad, d = h_pad.shape
  c = wc.shape[1]
  return pl.pallas_call(
      functools.partial(_pool_body, n_groups),
      in_specs=[
          pl.BlockSpec((npad, d), lambda: (0, 0)),
          pl.BlockSpec((n_groups, npad), lambda: (0, 0)),
          pl.BlockSpec((d, c), lambda: (0, 0)),
          pl.BlockSpec((1, c), lambda: (0, 0)),
      ],
      out_specs=pl.BlockSpec((n_groups, c), lambda: (0, 0)),
      out_shape=jax.ShapeDtypeStruct((n_groups, c), jnp.float32),
  )(h_pad, jnp.broadcast_to(batch_pad[None, :], (n_groups, npad)), wc,
    bc.reshape(1, c))


def kernel(x, edge_index, edge_type, batch, W1, root1, b1, W2, root2, b2,
           Wc, bc):
  n, d = x.shape
  r = W1.shape[0]
  e = edge_index.shape[1]
  n_groups = NUM_GRAPHS

  src = edge_index[0].astype(jnp.int32)
  dst = edge_index[1].astype(jnp.int32)
  et = edge_type.astype(jnp.int32)
  batch32 = batch.astype(jnp.int32)

  ng = _ceil_to(-(-e // (NW * B)), 2 * CHUNK)
  e_pad = NW * B * ng
  n_acc = _ceil_to(n + 1, NS * B)
  nbins = n_acc * r
  pad = e_pad - e

  dummy_dst = n + jnp.arange(pad, dtype=jnp.int32) % (n_acc - n)
  src_p = jnp.concatenate([src, jnp.zeros((pad,), jnp.int32)]).reshape(-1, B)
  dst_p = jnp.concatenate([dst, dummy_dst]).reshape(-1, B)
  et_p = jnp.concatenate([et, jnp.zeros((pad,), jnp.int32)]).reshape(-1, B)

  w_edge, gidx = _make_sc_setup(n, r, ng, nbins)(dst_p, et_p, src_p)

  sc_layer = _make_sc_layer(n_acc, d, ng)
  n_blk = 1000

  def layer(h, w_rel, root, b):
    w_stack = jnp.concatenate([w_rel, root[None]], axis=0)
    xr = _relation_matmul(h, w_stack, n_blk)
    xr_flat = xr[:r].reshape(r * n, d)
    acc = sc_layer(xr_flat, gidx, dst_p, w_edge)
    acc0 = acc[:n]
    acc1 = acc[n_acc:n_acc + n]
    return _combine(acc0, acc1, xr[r], b, n_blk)

  h = layer(x, W1, root1, b1)
  h = layer(h, W2, root2, b2)

  n_pad = _ceil_to(n, B)
  h_pad = jnp.pad(h, ((0, n_pad - n), (0, 0)))
  batch_pad = jnp.concatenate(
      [batch32, jnp.full((n_pad - n,), n_groups, jnp.int32)])
  return _pool_classify(h_pad, batch_pad, Wc, bc, n_groups)


# speedup vs baseline: 1.1144x; 1.1142x over previous
import functools

import jax
import jax.numpy as jnp
from jax import lax
from jax.experimental import pallas as pl
from jax.experimental.pallas import tpu as pltpu
from jax.experimental.pallas import tpu_sc as plsc

NC = 2
NS = 16
NW = NC * NS
L = 16

B = 128
CHUNK = 8

NUM_GRAPHS = 64


def _ceil_to(a, m):
  return (a + m - 1) // m * m


def _sc_setup_body(n_nodes, n_rel, ng, nbins, dst_hbm, et_hbm, src_hbm,
                   w_hbm, gidx_hbm, cnt_sh, zbuf, cb0, cb1, etb, comp_all,
                   w_all, gidx_all, ones_v, sem_a, sem_b):
  s_id = lax.axis_index("s")
  c_id = lax.axis_index("c")
  wid = s_id * NC + c_id
  ng2 = 2 * ng

  bins_per_tile = nbins // NS
  def _z(i, _):
    zbuf[pl.ds(i * L, L)] = jnp.zeros((L,), jnp.float32)
    return 0
  lax.fori_loop(0, bins_per_tile // L, _z, 0)
  pltpu.sync_copy(zbuf, cnt_sh.at[pl.ds(s_id * bins_per_tile, bins_per_tile)])
  def _o(i, _):
    ones_v[pl.ds(i * L, L)] = jnp.ones((L,), jnp.float32)
    return 0
  lax.fori_loop(0, B // L, _o, 0)
  plsc.subcore_barrier()

  pltpu.sync_copy(dst_hbm.at[pl.ds(s_id * ng2, ng2)], cb0)
  pltpu.sync_copy(et_hbm.at[pl.ds(s_id * ng2, ng2)], cb1)
  def _comp(i, _):
    j = i // (B // L)
    k = i % (B // L)
    dv = cb0[j, pl.ds(k * L, L)]
    ev = cb1[j, pl.ds(k * L, L)]
    comp_all[j, pl.ds(k * L, L)] = dv * n_rel + ev
    return 0
  lax.fori_loop(0, ng2 * (B // L), _comp, 0)
  def _fire(g, _):
    pltpu.async_copy(ones_v, cnt_sh.at[comp_all.at[g]], sem_a, add=True)
    return 0
  lax.fori_loop(0, ng2, _fire, 0)
  def _drain(g, _):
    pltpu.make_async_copy(ones_v, cnt_sh.at[comp_all.at[g]], sem_a).wait()
    return 0
  lax.fori_loop(0, ng2, _drain, 0)
  plsc.subcore_barrier()

  pltpu.sync_copy(cnt_sh.at[pl.ds(s_id * bins_per_tile, bins_per_tile)], zbuf)
  def _r(i, _):
    v = zbuf[pl.ds(i * L, L)]
    zbuf[pl.ds(i * L, L)] = 1.0 / jnp.maximum(v, 1.0)
    return 0
  lax.fori_loop(0, bins_per_tile // L, _r, 0)
  pltpu.sync_copy(zbuf, cnt_sh.at[pl.ds(s_id * bins_per_tile, bins_per_tile)])
  plsc.subcore_barrier()

  pltpu.sync_copy(src_hbm.at[pl.ds(wid * ng, ng)], cb0.at[pl.ds(0, ng)])
  pltpu.sync_copy(dst_hbm.at[pl.ds(wid * ng, ng)], cb1.at[pl.ds(0, ng)])
  pltpu.sync_copy(et_hbm.at[pl.ds(wid * ng, ng)], etb)
  def _gix(i, _):
    j = i // (B // L)
    k = i % (B // L)
    sv = cb0[j, pl.ds(k * L, L)]
    dv = cb1[j, pl.ds(k * L, L)]
    ev = etb[j, pl.ds(k * L, L)]
    comp_all[j, pl.ds(k * L, L)] = dv * n_rel + ev
    gidx_all[j, pl.ds(k * L, L)] = ev * n_nodes + sv
    return 0
  lax.fori_loop(0, ng * (B // L), _gix, 0)
  def _wfire(g, _):
    pltpu.async_copy(cnt_sh.at[comp_all.at[g]], w_all.at[g], sem_b)
    return 0
  lax.fori_loop(0, ng, _wfire, 0)
  def _wdrain(g, _):
    pltpu.make_async_copy(cnt_sh.at[comp_all.at[g]], w_all.at[g], sem_b).wait()
    return 0
  lax.fori_loop(0, ng, _wdrain, 0)
  pltpu.sync_copy(w_all, w_hbm.at[pl.ds(wid * ng, ng)])
  pltpu.sync_copy(gidx_all, gidx_hbm.at[pl.ds(wid * ng, ng)])


def _make_sc_setup(n_nodes, n_rel, ng, nbins):
  mesh = plsc.VectorSubcoreMesh(core_axis_name="c", subcore_axis_name="s")
  body = functools.partial(_sc_setup_body, n_nodes, n_rel, ng, nbins)
  return pl.kernel(
      body,
      out_type=(
          jax.ShapeDtypeStruct((NW * ng, B), jnp.float32),
          jax.ShapeDtypeStruct((NW * ng, B), jnp.int32),
      ),
      mesh=mesh,
      scratch_types=[
          pltpu.VMEM_SHARED((nbins,), jnp.float32),
          pltpu.VMEM((nbins // NS,), jnp.float32),
          pltpu.VMEM((2 * ng, B), jnp.int32),
          pltpu.VMEM((2 * ng, B), jnp.int32),
          pltpu.VMEM((ng, B), jnp.int32),
          pltpu.VMEM((2 * ng, B), jnp.int32),
          pltpu.VMEM((ng, B), jnp.float32),
          pltpu.VMEM((ng, B), jnp.int32),
          pltpu.VMEM((B,), jnp.float32),
          pltpu.SemaphoreType.DMA,
          pltpu.SemaphoreType.DMA,
      ],
      compiler_params=pltpu.CompilerParams(needs_layout_passes=False),
      name="rgcn_sc_setup",
  )


def _sc_layer_body(n_acc, d, m0, m1, xr_hbm, gidx_hbm, dst_hbm, w_hbm,
                   out_hbm, acc_sh, r0, r1, g0, g1, d0, d1, w0, w1,
                   sg0, sg1, ss0, ss1, se0, se1):
  s_id = lax.axis_index("s")
  c_id = lax.axis_index("c")
  rows = (r0, r1)
  gts = (g0, g1)
  dts = (d0, d1)
  wts = (w0, w1)
  sgs = (sg0, sg1)
  sss = (ss0, ss1)
  ses = (se0, se1)
  m = jnp.where(c_id == 0, m0, m1)
  gbase = jnp.where(c_id == 0, s_id * m0, NS * m0 + s_id * m1)
  nchunks = m // CHUNK

  rows_per_tile = n_acc // NS
  def _z(i, _):
    for c8 in range(d // L):
      r0[i, pl.ds(c8 * L, L)] = jnp.zeros((L,), jnp.float32)
    return 0
  lax.fori_loop(0, B, _z, 0)
  for k in range(rows_per_tile // B):
    pltpu.sync_copy(r0, acc_sh.at[pl.ds(s_id * rows_per_tile + k * B, B)])

  def _ech_start(cc, p):
    off = pl.multiple_of(gbase + cc * CHUNK, 8)
    pltpu.async_copy(gidx_hbm.at[pl.ds(off, CHUNK)], gts[p], ses[p])
    pltpu.async_copy(dst_hbm.at[pl.ds(off, CHUNK)], dts[p], ses[p])
    pltpu.async_copy(w_hbm.at[pl.ds(off, CHUNK)], wts[p], ses[p])

  def _ech_wait(p):
    base = pl.multiple_of(gbase, 8)
    pltpu.make_async_copy(gidx_hbm.at[pl.ds(base, CHUNK)], gts[p],
                          ses[p]).wait()
    pltpu.make_async_copy(dst_hbm.at[pl.ds(base, CHUNK)], dts[p],
                          ses[p]).wait()
    pltpu.make_async_copy(w_hbm.at[pl.ds(base, CHUNK)], wts[p],
                          ses[p]).wait()

  _ech_start(0, 0)
  plsc.subcore_barrier()

  def _scale(buf, wt, gg):
    def _body(j, _):
      wv = wt[gg, pl.ds(j * L, L)]
      for k in range(L):
        w = wv[k]
        i = j * L + k
        for c8 in range(d // L):
          buf[i, pl.ds(c8 * L, L)] = buf[i, pl.ds(c8 * L, L)] * w
      return 0
    lax.fori_loop(0, B // L, _body, 0)

  def _gwait(b):
    pltpu.make_async_copy(xr_hbm.at[gts[0].at[0]], rows[b], sgs[b]).wait()

  def _swait(b):
    pltpu.make_async_copy(rows[b], acc_sh.at[dts[0].at[0]], sss[b]).wait()

  def _chunk(cc, p):
    _ech_wait(p)
    @pl.when(cc + 1 < nchunks)
    def _pref():
      _ech_start(cc + 1, 1 - p)
    for pair in range(CHUNK // 2):
      gg0 = 2 * pair
      gg1 = 2 * pair + 1
      glob0 = cc * CHUNK + gg0
      @pl.when(glob0 >= 2)
      def _w0():
        _swait(0)
        _swait(1)
      pltpu.async_copy(xr_hbm.at[gts[p].at[gg0]], rows[0], sgs[0])
      pltpu.async_copy(xr_hbm.at[gts[p].at[gg1]], rows[1], sgs[1])
      _gwait(0)
      _scale(r0, wts[p], gg0)
      pltpu.async_copy(r0, acc_sh.at[dts[p].at[gg0]], sss[0], add=True)
      _gwait(1)
      _scale(r1, wts[p], gg1)
      pltpu.async_copy(r1, acc_sh.at[dts[p].at[gg1]], sss[1], add=True)

  def _iter(t, _):
    _chunk(2 * t, 0)
    _chunk(2 * t + 1, 1)
    return 0
  lax.fori_loop(0, nchunks // 2, _iter, 0)
  _swait(0)
  _swait(1)
  plsc.subcore_barrier()

  base = pl.multiple_of(c_id * n_acc + s_id * rows_per_tile, 8)
  pltpu.sync_copy(acc_sh.at[pl.ds(s_id * rows_per_tile, rows_per_tile)],
                  out_hbm.at[pl.ds(base, rows_per_tile)])


def _make_sc_layer(n_acc, d, m0, m1):
  mesh = plsc.VectorSubcoreMesh(core_axis_name="c", subcore_axis_name="s")
  body = functools.partial(_sc_layer_body, n_acc, d, m0, m1)
  return pl.kernel(
      body,
      out_type=jax.ShapeDtypeStruct((NC * n_acc, d), jnp.float32),
      mesh=mesh,
      scratch_types=[
          pltpu.VMEM_SHARED((n_acc, d), jnp.float32),
          pltpu.VMEM((B, d), jnp.float32),
          pltpu.VMEM((B, d), jnp.float32),
          pltpu.VMEM((CHUNK, B), jnp.int32),
          pltpu.VMEM((CHUNK, B), jnp.int32),
          pltpu.VMEM((CHUNK, B), jnp.int32),
          pltpu.VMEM((CHUNK, B), jnp.int32),
          pltpu.VMEM((CHUNK, B), jnp.float32),
          pltpu.VMEM((CHUNK, B), jnp.float32),
          pltpu.SemaphoreType.DMA,
          pltpu.SemaphoreType.DMA,
          pltpu.SemaphoreType.DMA,
          pltpu.SemaphoreType.DMA,
          pltpu.SemaphoreType.DMA,
          pltpu.SemaphoreType.DMA,
      ],
      name="rgcn_sc_layer",
  )


def _mm_body(x_ref, w_ref, o_ref):
  o_ref[0] = jnp.dot(x_ref[...], w_ref[0],
                     preferred_element_type=jnp.float32)


def _relation_matmul(x, w_stack, n_blk):
  n, d = x.shape
  r1 = w_stack.shape[0]
  grid = (r1, n // n_blk)
  return pl.pallas_call(
      _mm_body,
      grid=grid,
      in_specs=[
          pl.BlockSpec((n_blk, d), lambda r, i: (i, 0)),
          pl.BlockSpec((1, d, d), lambda r, i: (r, 0, 0)),
      ],
      out_specs=pl.BlockSpec((1, n_blk, d), lambda r, i: (r, i, 0)),
      out_shape=jax.ShapeDtypeStruct((r1, n, d), jnp.float32),
  )(x, w_stack)


def _combine_body(a0_ref, a1_ref, xr_ref, b_ref, o_ref):
  o_ref[...] = jnp.maximum(
      a0_ref[...] + a1_ref[...] + xr_ref[...] + b_ref[...], 0.0)


def _combine(acc0, acc1, xr_root, b, n_blk):
  n, d = acc0.shape
  grid = (n // n_blk,)
  return pl.pallas_call(
      _combine_body,
      grid=grid,
      in_specs=[
          pl.BlockSpec((n_blk, d), lambda i: (i, 0)),
          pl.BlockSpec((n_blk, d), lambda i: (i, 0)),
          pl.BlockSpec((n_blk, d), lambda i: (i, 0)),
          pl.BlockSpec((1, d), lambda i: (0, 0)),
      ],
      out_specs=pl.BlockSpec((n_blk, d), lambda i: (i, 0)),
      out_shape=jax.ShapeDtypeStruct((n, d), jnp.float32),
  )(acc0, acc1, xr_root, b.reshape(1, d))


def _pool_body(n_groups, h_ref, batch_ref, wc_ref, bc_ref, o_ref):
  npad = h_ref.shape[0]
  ids = lax.broadcasted_iota(jnp.int32, (n_groups, npad), 0)
  onehot = jnp.where(ids == batch_ref[...], 1.0, 0.0)
  sums = jnp.dot(onehot, h_ref[...], preferred_element_type=jnp.float32)
  cnt = jnp.sum(onehot, axis=1, keepdims=True)
  g = sums / jnp.maximum(cnt, 1.0)
  o_ref[...] = jnp.dot(g, wc_ref[...],
                       preferred_element_type=jnp.float32) + bc_ref[...]


def _pool_classify(h_pad, batch_pad, wc, bc, n_groups):
  npad, d = h_pad.shape
  c = wc.shape[1]
  return pl.pallas_call(
      functools.partial(_pool_body, n_groups),
      in_specs=[
          pl.BlockSpec((npad, d), lambda: (0, 0)),
          pl.BlockSpec((n_groups, npad), lambda: (0, 0)),
          pl.BlockSpec((d, c), lambda: (0, 0)),
          pl.BlockSpec((1, c), lambda: (0, 0)),
      ],
      out_specs=pl.BlockSpec((n_groups, c), lambda: (0, 0)),
      out_shape=jax.ShapeDtypeStruct((n_groups, c), jnp.float32),
  )(h_pad, jnp.broadcast_to(batch_pad[None, :], (n_groups, npad)), wc,
    bc.reshape(1, c))


def kernel(x, edge_index, edge_type, batch, W1, root1, b1, W2, root2, b2,
           Wc, bc):
  n, d = x.shape
  r = W1.shape[0]
  e = edge_index.shape[1]
  n_groups = NUM_GRAPHS

  src = edge_index[0].astype(jnp.int32)
  dst = edge_index[1].astype(jnp.int32)
  et = edge_type.astype(jnp.int32)
  batch32 = batch.astype(jnp.int32)

  ng = _ceil_to(-(-e // (NW * B)), 2 * CHUNK)
  e_pad = NW * B * ng
  n_acc = _ceil_to(n + 1, NS * B)
  nbins = n_acc * r
  pad = e_pad - e

  dummy_dst = n + jnp.arange(pad, dtype=jnp.int32) % (n_acc - n)
  src_p = jnp.concatenate([src, jnp.zeros((pad,), jnp.int32)]).reshape(-1, B)
  dst_p = jnp.concatenate([dst, dummy_dst]).reshape(-1, B)
  et_p = jnp.concatenate([et, jnp.zeros((pad,), jnp.int32)]).reshape(-1, B)

  w_edge, gidx = _make_sc_setup(n, r, ng, nbins)(dst_p, et_p, src_p)

  m0, m1 = 112, 48
  sc_layer = _make_sc_layer(n_acc, d, m0, m1)
  n_blk = 1000

  def layer(h, w_rel, root, b):
    w_stack = jnp.concatenate([w_rel, root[None]], axis=0)
    xr = _relation_matmul(h, w_stack, n_blk)
    xr_flat = xr[:r].reshape(r * n, d)
    acc = sc_layer(xr_flat, gidx, dst_p, w_edge)
    acc0 = acc[:n]
    acc1 = acc[n_acc:n_acc + n]
    return _combine(acc0, acc1, xr[r], b, n_blk)

  h = layer(x, W1, root1, b1)
  h = layer(h, W2, root2, b2)

  n_pad = _ceil_to(n, B)
  h_pad = jnp.pad(h, ((0, n_pad - n), (0, 0)))
  batch_pad = jnp.concatenate(
      [batch32, jnp.full((n_pad - n,), n_groups, jnp.int32)])
  return _pool_classify(h_pad, batch_pad, Wc, bc, n_groups)


# speedup vs baseline: 1.1891x; 1.0670x over previous
import functools

import jax
import jax.numpy as jnp
from jax import lax
from jax.experimental import pallas as pl
from jax.experimental.pallas import tpu as pltpu
from jax.experimental.pallas import tpu_sc as plsc

NC = 2
NS = 16
NW = NC * NS
L = 16

B = 128
CHUNK = 8

NUM_GRAPHS = 64


def _ceil_to(a, m):
  return (a + m - 1) // m * m


def _sc_setup_body(n_nodes, n_rel, ng, nbins, dst_hbm, et_hbm, src_hbm,
                   w_hbm, gidx_hbm, cnt_sh, zbuf, cb0, cb1, etb, comp_all,
                   w_all, gidx_all, ones_v, sem_a, sem_b):
  s_id = lax.axis_index("s")
  c_id = lax.axis_index("c")
  wid = s_id * NC + c_id
  ng2 = 2 * ng

  bins_per_tile = nbins // NS
  def _z(i, _):
    zbuf[pl.ds(i * L, L)] = jnp.zeros((L,), jnp.float32)
    return 0
  lax.fori_loop(0, bins_per_tile // L, _z, 0)
  pltpu.sync_copy(zbuf, cnt_sh.at[pl.ds(s_id * bins_per_tile, bins_per_tile)])
  def _o(i, _):
    ones_v[pl.ds(i * L, L)] = jnp.ones((L,), jnp.float32)
    return 0
  lax.fori_loop(0, B // L, _o, 0)
  plsc.subcore_barrier()

  pltpu.sync_copy(dst_hbm.at[pl.ds(s_id * ng2, ng2)], cb0)
  pltpu.sync_copy(et_hbm.at[pl.ds(s_id * ng2, ng2)], cb1)
  def _comp(i, _):
    j = i // (B // L)
    k = i % (B // L)
    dv = cb0[j, pl.ds(k * L, L)]
    ev = cb1[j, pl.ds(k * L, L)]
    comp_all[j, pl.ds(k * L, L)] = dv * n_rel + ev
    return 0
  lax.fori_loop(0, ng2 * (B // L), _comp, 0)
  def _fire(g, _):
    pltpu.async_copy(ones_v, cnt_sh.at[comp_all.at[g]], sem_a, add=True)
    return 0
  lax.fori_loop(0, ng2, _fire, 0)
  def _drain(g, _):
    pltpu.make_async_copy(ones_v, cnt_sh.at[comp_all.at[g]], sem_a).wait()
    return 0
  lax.fori_loop(0, ng2, _drain, 0)
  plsc.subcore_barrier()

  pltpu.sync_copy(cnt_sh.at[pl.ds(s_id * bins_per_tile, bins_per_tile)], zbuf)
  def _r(i, _):
    v = zbuf[pl.ds(i * L, L)]
    zbuf[pl.ds(i * L, L)] = 1.0 / jnp.maximum(v, 1.0)
    return 0
  lax.fori_loop(0, bins_per_tile // L, _r, 0)
  pltpu.sync_copy(zbuf, cnt_sh.at[pl.ds(s_id * bins_per_tile, bins_per_tile)])
  plsc.subcore_barrier()

  pltpu.sync_copy(src_hbm.at[pl.ds(wid * ng, ng)], cb0.at[pl.ds(0, ng)])
  pltpu.sync_copy(dst_hbm.at[pl.ds(wid * ng, ng)], cb1.at[pl.ds(0, ng)])
  pltpu.sync_copy(et_hbm.at[pl.ds(wid * ng, ng)], etb)
  def _gix(i, _):
    j = i // (B // L)
    k = i % (B // L)
    sv = cb0[j, pl.ds(k * L, L)]
    dv = cb1[j, pl.ds(k * L, L)]
    ev = etb[j, pl.ds(k * L, L)]
    comp_all[j, pl.ds(k * L, L)] = dv * n_rel + ev
    gidx_all[j, pl.ds(k * L, L)] = ev * n_nodes + sv
    return 0
  lax.fori_loop(0, ng * (B // L), _gix, 0)
  def _wfire(g, _):
    pltpu.async_copy(cnt_sh.at[comp_all.at[g]], w_all.at[g], sem_b)
    return 0
  lax.fori_loop(0, ng, _wfire, 0)
  def _wdrain(g, _):
    pltpu.make_async_copy(cnt_sh.at[comp_all.at[g]], w_all.at[g], sem_b).wait()
    return 0
  lax.fori_loop(0, ng, _wdrain, 0)
  pltpu.sync_copy(w_all, w_hbm.at[pl.ds(wid * ng, ng)])
  pltpu.sync_copy(gidx_all, gidx_hbm.at[pl.ds(wid * ng, ng)])


def _make_sc_setup(n_nodes, n_rel, ng, nbins):
  mesh = plsc.VectorSubcoreMesh(core_axis_name="c", subcore_axis_name="s")
  body = functools.partial(_sc_setup_body, n_nodes, n_rel, ng, nbins)
  return pl.kernel(
      body,
      out_type=(
          jax.ShapeDtypeStruct((NW * ng, B), jnp.float32),
          jax.ShapeDtypeStruct((NW * ng, B), jnp.int32),
      ),
      mesh=mesh,
      scratch_types=[
          pltpu.VMEM_SHARED((nbins,), jnp.float32),
          pltpu.VMEM((nbins // NS,), jnp.float32),
          pltpu.VMEM((2 * ng, B), jnp.int32),
          pltpu.VMEM((2 * ng, B), jnp.int32),
          pltpu.VMEM((ng, B), jnp.int32),
          pltpu.VMEM((2 * ng, B), jnp.int32),
          pltpu.VMEM((ng, B), jnp.float32),
          pltpu.VMEM((ng, B), jnp.int32),
          pltpu.VMEM((B,), jnp.float32),
          pltpu.SemaphoreType.DMA,
          pltpu.SemaphoreType.DMA,
      ],
      compiler_params=pltpu.CompilerParams(needs_layout_passes=False),
      name="rgcn_sc_setup",
  )


def _sc_layer_body(n_acc, d, m0, m1, xr_hbm, gidx_hbm, dst_hbm, w_hbm,
                   out_hbm, acc_sh, r0, r1, g0, g1, d0, d1, w0, w1,
                   sg0, sg1, ss0, ss1, se0, se1):
  s_id = lax.axis_index("s")
  c_id = lax.axis_index("c")
  rows = (r0, r1)
  gts = (g0, g1)
  dts = (d0, d1)
  wts = (w0, w1)
  sgs = (sg0, sg1)
  sss = (ss0, ss1)
  ses = (se0, se1)
  m = jnp.where(c_id == 0, m0, m1)
  gbase = jnp.where(c_id == 0, s_id * m0, NS * m0 + s_id * m1)
  nchunks = m // CHUNK

  rows_per_tile = n_acc // NS
  def _z(i, _):
    for c8 in range(d // L):
      r0[i, pl.ds(c8 * L, L)] = jnp.zeros((L,), jnp.float32)
    return 0
  lax.fori_loop(0, B, _z, 0)
  for k in range(rows_per_tile // B):
    pltpu.sync_copy(r0, acc_sh.at[pl.ds(s_id * rows_per_tile + k * B, B)])

  def _ech_start(cc, p):
    off = pl.multiple_of(gbase + cc * CHUNK, 8)
    pltpu.async_copy(gidx_hbm.at[pl.ds(off, CHUNK)], gts[p], ses[p])
    pltpu.async_copy(dst_hbm.at[pl.ds(off, CHUNK)], dts[p], ses[p])
    pltpu.async_copy(w_hbm.at[pl.ds(off, CHUNK)], wts[p], ses[p])

  def _ech_wait(p):
    base = pl.multiple_of(gbase, 8)
    pltpu.make_async_copy(gidx_hbm.at[pl.ds(base, CHUNK)], gts[p],
                          ses[p]).wait()
    pltpu.make_async_copy(dst_hbm.at[pl.ds(base, CHUNK)], dts[p],
                          ses[p]).wait()
    pltpu.make_async_copy(w_hbm.at[pl.ds(base, CHUNK)], wts[p],
                          ses[p]).wait()

  _ech_start(0, 0)
  plsc.subcore_barrier()

  def _scale(buf, wt, gg):
    def _body(j, _):
      wv = wt[gg, pl.ds(j * L, L)]
      for k in range(L):
        w = wv[k]
        i = j * L + k
        for c8 in range(d // L):
          buf[i, pl.ds(c8 * L, L)] = buf[i, pl.ds(c8 * L, L)] * w
      return 0
    lax.fori_loop(0, B // L, _body, 0)

  def _gwait(b):
    pltpu.make_async_copy(xr_hbm.at[gts[0].at[0]], rows[b], sgs[b]).wait()

  def _swait(b):
    pltpu.make_async_copy(rows[b], acc_sh.at[dts[0].at[0]], sss[b]).wait()

  def _chunk(cc, p):
    _ech_wait(p)
    @pl.when(cc + 1 < nchunks)
    def _pref():
      _ech_start(cc + 1, 1 - p)
    for pair in range(CHUNK // 2):
      gg0 = 2 * pair
      gg1 = 2 * pair + 1
      glob0 = cc * CHUNK + gg0
      @pl.when(glob0 >= 2)
      def _w0():
        _swait(0)
        _swait(1)
      pltpu.async_copy(xr_hbm.at[gts[p].at[gg0]], rows[0], sgs[0])
      pltpu.async_copy(xr_hbm.at[gts[p].at[gg1]], rows[1], sgs[1])
      _gwait(0)
      _scale(r0, wts[p], gg0)
      pltpu.async_copy(r0, acc_sh.at[dts[p].at[gg0]], sss[0], add=True)
      _gwait(1)
      _scale(r1, wts[p], gg1)
      pltpu.async_copy(r1, acc_sh.at[dts[p].at[gg1]], sss[1], add=True)

  def _iter(t, _):
    _chunk(2 * t, 0)
    _chunk(2 * t + 1, 1)
    return 0
  lax.fori_loop(0, nchunks // 2, _iter, 0)
  _swait(0)
  _swait(1)
  plsc.subcore_barrier()

  base = pl.multiple_of(c_id * n_acc + s_id * rows_per_tile, 8)
  pltpu.sync_copy(acc_sh.at[pl.ds(s_id * rows_per_tile, rows_per_tile)],
                  out_hbm.at[pl.ds(base, rows_per_tile)])


def _make_sc_layer(n_acc, d, m0, m1):
  mesh = plsc.VectorSubcoreMesh(core_axis_name="c", subcore_axis_name="s")
  body = functools.partial(_sc_layer_body, n_acc, d, m0, m1)
  return pl.kernel(
      body,
      out_type=jax.ShapeDtypeStruct((NC * n_acc, d), jnp.float32),
      mesh=mesh,
      scratch_types=[
          pltpu.VMEM_SHARED((n_acc, d), jnp.float32),
          pltpu.VMEM((B, d), jnp.float32),
          pltpu.VMEM((B, d), jnp.float32),
          pltpu.VMEM((CHUNK, B), jnp.int32),
          pltpu.VMEM((CHUNK, B), jnp.int32),
          pltpu.VMEM((CHUNK, B), jnp.int32),
          pltpu.VMEM((CHUNK, B), jnp.int32),
          pltpu.VMEM((CHUNK, B), jnp.float32),
          pltpu.VMEM((CHUNK, B), jnp.float32),
          pltpu.SemaphoreType.DMA,
          pltpu.SemaphoreType.DMA,
          pltpu.SemaphoreType.DMA,
          pltpu.SemaphoreType.DMA,
          pltpu.SemaphoreType.DMA,
          pltpu.SemaphoreType.DMA,
      ],
      name="rgcn_sc_layer",
  )


def _mm_body(x_ref, w_ref, o_ref):
  o_ref[0] = jnp.dot(x_ref[...], w_ref[0],
                     preferred_element_type=jnp.float32)


def _relation_matmul(x, w_stack, n_blk):
  n, d = x.shape
  r1 = w_stack.shape[0]
  grid = (r1, n // n_blk)
  return pl.pallas_call(
      _mm_body,
      grid=grid,
      in_specs=[
          pl.BlockSpec((n_blk, d), lambda r, i: (i, 0)),
          pl.BlockSpec((1, d, d), lambda r, i: (r, 0, 0)),
      ],
      out_specs=pl.BlockSpec((1, n_blk, d), lambda r, i: (r, i, 0)),
      out_shape=jax.ShapeDtypeStruct((r1, n, d), jnp.float32),
  )(x, w_stack)


def _combine_body(a0_ref, a1_ref, xr_ref, b_ref, o_ref):
  o_ref[...] = jnp.maximum(
      a0_ref[...] + a1_ref[...] + xr_ref[...] + b_ref[...], 0.0)


def _combine(acc0, acc1, xr_root, b, n_blk):
  n, d = acc0.shape
  grid = (n // n_blk,)
  return pl.pallas_call(
      _combine_body,
      grid=grid,
      in_specs=[
          pl.BlockSpec((n_blk, d), lambda i: (i, 0)),
          pl.BlockSpec((n_blk, d), lambda i: (i, 0)),
          pl.BlockSpec((n_blk, d), lambda i: (i, 0)),
          pl.BlockSpec((1, d), lambda i: (0, 0)),
      ],
      out_specs=pl.BlockSpec((n_blk, d), lambda i: (i, 0)),
      out_shape=jax.ShapeDtypeStruct((n, d), jnp.float32),
  )(acc0, acc1, xr_root, b.reshape(1, d))


def _pool_body(n_groups, h_ref, batch_ref, wc_ref, bc_ref, o_ref):
  npad = h_ref.shape[0]
  ids = lax.broadcasted_iota(jnp.int32, (n_groups, npad), 0)
  onehot = jnp.where(ids == batch_ref[...], 1.0, 0.0)
  sums = jnp.dot(onehot, h_ref[...], preferred_element_type=jnp.float32)
  cnt = jnp.sum(onehot, axis=1, keepdims=True)
  g = sums / jnp.maximum(cnt, 1.0)
  o_ref[...] = jnp.dot(g, wc_ref[...],
                       preferred_element_type=jnp.float32) + bc_ref[...]


def _pool_classify(h_pad, batch_pad, wc, bc, n_groups):
  npad, d = h_pad.shape
  c = wc.shape[1]
  return pl.pallas_call(
      functools.partial(_pool_body, n_groups),
      in_specs=[
          pl.BlockSpec((npad, d), lambda: (0, 0)),
          pl.BlockSpec((n_groups, npad), lambda: (0, 0)),
          pl.BlockSpec((d, c), lambda: (0, 0)),
          pl.BlockSpec((1, c), lambda: (0, 0)),
      ],
      out_specs=pl.BlockSpec((n_groups, c), lambda: (0, 0)),
      out_shape=jax.ShapeDtypeStruct((n_groups, c), jnp.float32),
  )(h_pad, jnp.broadcast_to(batch_pad[None, :], (n_groups, npad)), wc,
    bc.reshape(1, c))


def kernel(x, edge_index, edge_type, batch, W1, root1, b1, W2, root2, b2,
           Wc, bc):
  n, d = x.shape
  r = W1.shape[0]
  e = edge_index.shape[1]
  n_groups = NUM_GRAPHS

  src = edge_index[0].astype(jnp.int32)
  dst = edge_index[1].astype(jnp.int32)
  et = edge_type.astype(jnp.int32)
  batch32 = batch.astype(jnp.int32)

  ng = _ceil_to(-(-e // (NW * B)), 2 * CHUNK)
  e_pad = NW * B * ng
  n_acc = _ceil_to(n + 1, NS * B)
  nbins = n_acc * r
  pad = e_pad - e

  dummy_dst = n + jnp.arange(pad, dtype=jnp.int32) % (n_acc - n)
  src_p = jnp.concatenate([src, jnp.zeros((pad,), jnp.int32)]).reshape(-1, B)
  dst_p = jnp.concatenate([dst, dummy_dst]).reshape(-1, B)
  et_p = jnp.concatenate([et, jnp.zeros((pad,), jnp.int32)]).reshape(-1, B)

  w_edge, gidx = _make_sc_setup(n, r, ng, nbins)(dst_p, et_p, src_p)

  m0, m1 = 128, 32
  sc_layer = _make_sc_layer(n_acc, d, m0, m1)
  n_blk = 1000

  def layer(h, w_rel, root, b):
    w_stack = jnp.concatenate([w_rel, root[None]], axis=0)
    xr = _relation_matmul(h, w_stack, n_blk)
    xr_flat = xr[:r].reshape(r * n, d)
    acc = sc_layer(xr_flat, gidx, dst_p, w_edge)
    acc0 = acc[:n]
    acc1 = acc[n_acc:n_acc + n]
    return _combine(acc0, acc1, xr[r], b, n_blk)

  h = layer(x, W1, root1, b1)
  h = layer(h, W2, root2, b2)

  n_pad = _ceil_to(n, B)
  h_pad = jnp.pad(h, ((0, n_pad - n), (0, 0)))
  batch_pad = jnp.concatenate(
      [batch32, jnp.full((n_pad - n,), n_groups, jnp.int32)])
  return _pool_classify(h_pad, batch_pad, Wc, bc, n_groups)


# speedup vs baseline: 1.3683x; 1.1507x over previous
import functools

import jax
import jax.numpy as jnp
from jax import lax
from jax.experimental import pallas as pl
from jax.experimental.pallas import tpu as pltpu
from jax.experimental.pallas import tpu_sc as plsc

NC = 2
NS = 16
NW = NC * NS
L = 16

B = 128
CHUNK = 8

NUM_GRAPHS = 64


def _ceil_to(a, m):
  return (a + m - 1) // m * m


def _sc_setup_body(n_nodes, n_rel, ng, nbins, dst_hbm, et_hbm, src_hbm,
                   w_hbm, gidx_hbm, cnt_sh, zbuf, cb0, cb1, etb, comp_all,
                   w_all, gidx_all, ones_v, sem_a, sem_b):
  s_id = lax.axis_index("s")
  c_id = lax.axis_index("c")
  wid = s_id * NC + c_id
  ng2 = 2 * ng

  bins_per_tile = nbins // NS
  def _z(i, _):
    zbuf[pl.ds(i * L, L)] = jnp.zeros((L,), jnp.float32)
    return 0
  lax.fori_loop(0, bins_per_tile // L, _z, 0)
  pltpu.sync_copy(zbuf, cnt_sh.at[pl.ds(s_id * bins_per_tile, bins_per_tile)])
  def _o(i, _):
    ones_v[pl.ds(i * L, L)] = jnp.ones((L,), jnp.float32)
    return 0
  lax.fori_loop(0, B // L, _o, 0)
  plsc.subcore_barrier()

  pltpu.sync_copy(dst_hbm.at[pl.ds(s_id * ng2, ng2)], cb0)
  pltpu.sync_copy(et_hbm.at[pl.ds(s_id * ng2, ng2)], cb1)
  def _comp(i, _):
    j = i // (B // L)
    k = i % (B // L)
    dv = cb0[j, pl.ds(k * L, L)]
    ev = cb1[j, pl.ds(k * L, L)]
    comp_all[j, pl.ds(k * L, L)] = dv * n_rel + ev
    return 0
  lax.fori_loop(0, ng2 * (B // L), _comp, 0)
  def _fire(g, _):
    pltpu.async_copy(ones_v, cnt_sh.at[comp_all.at[g]], sem_a, add=True)
    return 0
  lax.fori_loop(0, ng2, _fire, 0)
  def _drain(g, _):
    pltpu.make_async_copy(ones_v, cnt_sh.at[comp_all.at[g]], sem_a).wait()
    return 0
  lax.fori_loop(0, ng2, _drain, 0)
  plsc.subcore_barrier()

  pltpu.sync_copy(cnt_sh.at[pl.ds(s_id * bins_per_tile, bins_per_tile)], zbuf)
  def _r(i, _):
    v = zbuf[pl.ds(i * L, L)]
    zbuf[pl.ds(i * L, L)] = 1.0 / jnp.maximum(v, 1.0)
    return 0
  lax.fori_loop(0, bins_per_tile // L, _r, 0)
  pltpu.sync_copy(zbuf, cnt_sh.at[pl.ds(s_id * bins_per_tile, bins_per_tile)])
  plsc.subcore_barrier()

  pltpu.sync_copy(src_hbm.at[pl.ds(wid * ng, ng)], cb0.at[pl.ds(0, ng)])
  pltpu.sync_copy(dst_hbm.at[pl.ds(wid * ng, ng)], cb1.at[pl.ds(0, ng)])
  pltpu.sync_copy(et_hbm.at[pl.ds(wid * ng, ng)], etb)
  def _gix(i, _):
    j = i // (B // L)
    k = i % (B // L)
    sv = cb0[j, pl.ds(k * L, L)]
    dv = cb1[j, pl.ds(k * L, L)]
    ev = etb[j, pl.ds(k * L, L)]
    comp_all[j, pl.ds(k * L, L)] = dv * n_rel + ev
    gidx_all[j, pl.ds(k * L, L)] = ev * n_nodes + sv
    return 0
  lax.fori_loop(0, ng * (B // L), _gix, 0)
  def _wfire(g, _):
    pltpu.async_copy(cnt_sh.at[comp_all.at[g]], w_all.at[g], sem_b)
    return 0
  lax.fori_loop(0, ng, _wfire, 0)
  def _wdrain(g, _):
    pltpu.make_async_copy(cnt_sh.at[comp_all.at[g]], w_all.at[g], sem_b).wait()
    return 0
  lax.fori_loop(0, ng, _wdrain, 0)
  pltpu.sync_copy(w_all, w_hbm.at[pl.ds(wid * ng, ng)])
  pltpu.sync_copy(gidx_all, gidx_hbm.at[pl.ds(wid * ng, ng)])


def _make_sc_setup(n_nodes, n_rel, ng, nbins):
  mesh = plsc.VectorSubcoreMesh(core_axis_name="c", subcore_axis_name="s")
  body = functools.partial(_sc_setup_body, n_nodes, n_rel, ng, nbins)
  return pl.kernel(
      body,
      out_type=(
          jax.ShapeDtypeStruct((NW * ng, B), jnp.float32),
          jax.ShapeDtypeStruct((NW * ng, B), jnp.int32),
      ),
      mesh=mesh,
      scratch_types=[
          pltpu.VMEM_SHARED((nbins,), jnp.float32),
          pltpu.VMEM((nbins // NS,), jnp.float32),
          pltpu.VMEM((2 * ng, B), jnp.int32),
          pltpu.VMEM((2 * ng, B), jnp.int32),
          pltpu.VMEM((ng, B), jnp.int32),
          pltpu.VMEM((2 * ng, B), jnp.int32),
          pltpu.VMEM((ng, B), jnp.float32),
          pltpu.VMEM((ng, B), jnp.int32),
          pltpu.VMEM((B,), jnp.float32),
          pltpu.SemaphoreType.DMA,
          pltpu.SemaphoreType.DMA,
      ],
      compiler_params=pltpu.CompilerParams(needs_layout_passes=False),
      name="rgcn_sc_setup",
  )


def _sc_layer_body(n_acc, d, m0, m1, xr_hbm, gidx_hbm, dst_hbm, w_hbm,
                   out_hbm, acc_sh, r0, r1, g0, g1, d0, d1, w0, w1,
                   sg0, sg1, ss0, ss1, se0, se1):
  s_id = lax.axis_index("s")
  c_id = lax.axis_index("c")
  rows = (r0, r1)
  gts = (g0, g1)
  dts = (d0, d1)
  wts = (w0, w1)
  sgs = (sg0, sg1)
  sss = (ss0, ss1)
  ses = (se0, se1)
  m = jnp.where(c_id == 0, m0, m1)
  gbase = jnp.where(c_id == 0, s_id * m0, NS * m0 + s_id * m1)
  nchunks = m // CHUNK

  rows_per_tile = n_acc // NS
  def _z(i, _):
    for c8 in range(d // L):
      r0[i, pl.ds(c8 * L, L)] = jnp.zeros((L,), jnp.float32)
    return 0
  lax.fori_loop(0, B, _z, 0)
  for k in range(rows_per_tile // B):
    pltpu.sync_copy(r0, acc_sh.at[pl.ds(s_id * rows_per_tile + k * B, B)])

  def _ech_start(cc, p):
    off = pl.multiple_of(gbase + cc * CHUNK, 8)
    pltpu.async_copy(gidx_hbm.at[pl.ds(off, CHUNK)], gts[p], ses[p])
    pltpu.async_copy(dst_hbm.at[pl.ds(off, CHUNK)], dts[p], ses[p])
    pltpu.async_copy(w_hbm.at[pl.ds(off, CHUNK)], wts[p], ses[p])

  def _ech_wait(p):
    base = pl.multiple_of(gbase, 8)
    pltpu.make_async_copy(gidx_hbm.at[pl.ds(base, CHUNK)], gts[p],
                          ses[p]).wait()
    pltpu.make_async_copy(dst_hbm.at[pl.ds(base, CHUNK)], dts[p],
                          ses[p]).wait()
    pltpu.make_async_copy(w_hbm.at[pl.ds(base, CHUNK)], wts[p],
                          ses[p]).wait()

  _ech_start(0, 0)
  plsc.subcore_barrier()

  def _scale(buf, wt, gg):
    def _body(j, _):
      wv = wt[gg, pl.ds(j * L, L)]
      for k in range(L):
        w = wv[k]
        i = j * L + k
        for c8 in range(d // L):
          buf[i, pl.ds(c8 * L, L)] = buf[i, pl.ds(c8 * L, L)] * w
      return 0
    lax.fori_loop(0, B // L, _body, 0)

  def _gwait(b):
    pltpu.make_async_copy(xr_hbm.at[gts[0].at[0]], rows[b], sgs[b]).wait()

  def _swait(b):
    pltpu.make_async_copy(rows[b], acc_sh.at[dts[0].at[0]], sss[b]).wait()

  def _chunk(cc, p):
    _ech_wait(p)
    @pl.when(cc + 1 < nchunks)
    def _pref():
      _ech_start(cc + 1, 1 - p)
    for pair in range(CHUNK // 2):
      gg0 = 2 * pair
      gg1 = 2 * pair + 1
      glob0 = cc * CHUNK + gg0
      @pl.when(glob0 >= 2)
      def _w0():
        _swait(0)
        _swait(1)
      pltpu.async_copy(xr_hbm.at[gts[p].at[gg0]], rows[0], sgs[0])
      pltpu.async_copy(xr_hbm.at[gts[p].at[gg1]], rows[1], sgs[1])
      _gwait(0)
      _scale(r0, wts[p], gg0)
      pltpu.async_copy(r0, acc_sh.at[dts[p].at[gg0]], sss[0], add=True)
      _gwait(1)
      _scale(r1, wts[p], gg1)
      pltpu.async_copy(r1, acc_sh.at[dts[p].at[gg1]], sss[1], add=True)

  def _iter(t, _):
    _chunk(2 * t, 0)
    _chunk(2 * t + 1, 1)
    return 0
  lax.fori_loop(0, nchunks // 2, _iter, 0)
  _swait(0)
  _swait(1)
  plsc.subcore_barrier()

  base = pl.multiple_of(c_id * n_acc + s_id * rows_per_tile, 8)
  pltpu.sync_copy(acc_sh.at[pl.ds(s_id * rows_per_tile, rows_per_tile)],
                  out_hbm.at[pl.ds(base, rows_per_tile)])


def _make_sc_layer(n_acc, d, m0, m1):
  mesh = plsc.VectorSubcoreMesh(core_axis_name="c", subcore_axis_name="s")
  body = functools.partial(_sc_layer_body, n_acc, d, m0, m1)
  return pl.kernel(
      body,
      out_type=jax.ShapeDtypeStruct((NC * n_acc, d), jnp.float32),
      mesh=mesh,
      scratch_types=[
          pltpu.VMEM_SHARED((n_acc, d), jnp.float32),
          pltpu.VMEM((B, d), jnp.float32),
          pltpu.VMEM((B, d), jnp.float32),
          pltpu.VMEM((CHUNK, B), jnp.int32),
          pltpu.VMEM((CHUNK, B), jnp.int32),
          pltpu.VMEM((CHUNK, B), jnp.int32),
          pltpu.VMEM((CHUNK, B), jnp.int32),
          pltpu.VMEM((CHUNK, B), jnp.float32),
          pltpu.VMEM((CHUNK, B), jnp.float32),
          pltpu.SemaphoreType.DMA,
          pltpu.SemaphoreType.DMA,
          pltpu.SemaphoreType.DMA,
          pltpu.SemaphoreType.DMA,
          pltpu.SemaphoreType.DMA,
          pltpu.SemaphoreType.DMA,
      ],
      name="rgcn_sc_layer",
  )


def _mm_body(x_ref, w_ref, o_ref):
  o_ref[0] = jnp.dot(x_ref[...], w_ref[0],
                     preferred_element_type=jnp.float32)


def _relation_matmul(x, w_stack, n_blk):
  n, d = x.shape
  r1 = w_stack.shape[0]
  grid = (r1, n // n_blk)
  return pl.pallas_call(
      _mm_body,
      grid=grid,
      in_specs=[
          pl.BlockSpec((n_blk, d), lambda r, i: (i, 0)),
          pl.BlockSpec((1, d, d), lambda r, i: (r, 0, 0)),
      ],
      out_specs=pl.BlockSpec((1, n_blk, d), lambda r, i: (r, i, 0)),
      out_shape=jax.ShapeDtypeStruct((r1, n, d), jnp.float32),
  )(x, w_stack)


def _combine_body(acc_ref0, acc_ref1, xr_ref, b_ref, o_ref):
  o_ref[...] = jnp.maximum(
      acc_ref0[...] + acc_ref1[...] + xr_ref[0] + b_ref[...], 0.0)


def _combine(acc, xr, r, b, n_blk):
  n_acc2, d = acc.shape
  n_acc = n_acc2 // 2
  grid = (n_acc // n_blk,)
  nb = n_acc // n_blk
  return pl.pallas_call(
      _combine_body,
      grid=grid,
      in_specs=[
          pl.BlockSpec((n_blk, d), lambda i: (i, 0)),
          pl.BlockSpec((n_blk, d), lambda i: (i + nb, 0)),
          pl.BlockSpec((1, n_blk, d), lambda i: (r, i, 0)),
          pl.BlockSpec((1, d), lambda i: (0, 0)),
      ],
      out_specs=pl.BlockSpec((n_blk, d), lambda i: (i, 0)),
      out_shape=jax.ShapeDtypeStruct((n_acc, d), jnp.float32),
  )(acc, acc, xr, b.reshape(1, d))


def _pool_body(n_groups, h_ref, batch_ref, wc_ref, bc_ref, o_ref):
  npad = h_ref.shape[0]
  ids = lax.broadcasted_iota(jnp.int32, (n_groups, npad), 0)
  onehot = jnp.where(ids == batch_ref[...], 1.0, 0.0)
  sums = jnp.dot(onehot, h_ref[...], preferred_element_type=jnp.float32)
  cnt = jnp.sum(onehot, axis=1, keepdims=True)
  g = sums / jnp.maximum(cnt, 1.0)
  o_ref[...] = jnp.dot(g, wc_ref[...],
                       preferred_element_type=jnp.float32) + bc_ref[...]


def _pool_classify(h_pad, batch_pad, wc, bc, n_groups):
  npad, d = h_pad.shape
  c = wc.shape[1]
  return pl.pallas_call(
      functools.partial(_pool_body, n_groups),
      in_specs=[
          pl.BlockSpec((npad, d), lambda: (0, 0)),
          pl.BlockSpec((1, npad), lambda: (0, 0)),
          pl.BlockSpec((d, c), lambda: (0, 0)),
          pl.BlockSpec((1, c), lambda: (0, 0)),
      ],
      out_specs=pl.BlockSpec((n_groups, c), lambda: (0, 0)),
      out_shape=jax.ShapeDtypeStruct((n_groups, c), jnp.float32),
  )(h_pad, batch_pad.reshape(1, npad), wc, bc.reshape(1, c))


def kernel(x, edge_index, edge_type, batch, W1, root1, b1, W2, root2, b2,
           Wc, bc):
  n, d = x.shape
  r = W1.shape[0]
  e = edge_index.shape[1]
  n_groups = NUM_GRAPHS

  src = edge_index[0].astype(jnp.int32)
  dst = edge_index[1].astype(jnp.int32)
  et = edge_type.astype(jnp.int32)
  batch32 = batch.astype(jnp.int32)

  ng = _ceil_to(-(-e // (NW * B)), 2 * CHUNK)
  e_pad = NW * B * ng
  n_acc = _ceil_to(n + 1, NS * B)
  nbins = n_acc * r
  pad = e_pad - e

  dummy_dst = n + jnp.arange(pad, dtype=jnp.int32) % (n_acc - n)
  src_p = jnp.concatenate([src, jnp.zeros((pad,), jnp.int32)]).reshape(-1, B)
  dst_p = jnp.concatenate([dst, dummy_dst]).reshape(-1, B)
  et_p = jnp.concatenate([et, jnp.zeros((pad,), jnp.int32)]).reshape(-1, B)

  w_edge, gidx = _make_sc_setup(n_acc, r, ng, nbins)(dst_p, et_p, src_p)

  m0, m1 = 128, 32
  sc_layer = _make_sc_layer(n_acc, d, m0, m1)
  n_blk = 1024

  x_p = jnp.pad(x, ((0, n_acc - n), (0, 0)))

  def layer(h, w_rel, root, b):
    w_stack = jnp.concatenate([w_rel, root[None]], axis=0)
    xr = _relation_matmul(h, w_stack, n_blk)
    xr_flat = xr[:r].reshape(r * n_acc, d)
    acc = sc_layer(xr_flat, gidx, dst_p, w_edge)
    return _combine(acc, xr, r, b, n_blk)

  h = layer(x_p, W1, root1, b1)
  h = layer(h, W2, root2, b2)

  batch_pad = jnp.concatenate(
      [batch32, jnp.full((n_acc - n,), n_groups, jnp.int32)])
  return _pool_classify(h, batch_pad, Wc, bc, n_groups)


# speedup vs baseline: 1.3865x; 1.0133x over previous
import functools

import jax
import jax.numpy as jnp
from jax import lax
from jax.experimental import pallas as pl
from jax.experimental.pallas import tpu as pltpu
from jax.experimental.pallas import tpu_sc as plsc

NC = 2
NS = 16
NW = NC * NS
L = 16

B = 128
CHUNK = 8

NUM_GRAPHS = 64


def _ceil_to(a, m):
  return (a + m - 1) // m * m


def _sc_setup_body(n_nodes, n_rel, ng, nbins, dst_hbm, et_hbm, src_hbm,
                   w_hbm, gidx_hbm, cnt_sh, zbuf, cb0, cb1, etb, comp_all,
                   w_all, gidx_all, ones_v, sem_a, sem_b):
  s_id = lax.axis_index("s")
  c_id = lax.axis_index("c")
  wid = s_id * NC + c_id
  ng2 = 2 * ng

  bins_per_tile = nbins // NS
  def _z(i, _):
    zbuf[pl.ds(i * L, L)] = jnp.zeros((L,), jnp.float32)
    return 0
  lax.fori_loop(0, bins_per_tile // L, _z, 0)
  pltpu.sync_copy(zbuf, cnt_sh.at[pl.ds(s_id * bins_per_tile, bins_per_tile)])
  def _o(i, _):
    ones_v[pl.ds(i * L, L)] = jnp.ones((L,), jnp.float32)
    return 0
  lax.fori_loop(0, B // L, _o, 0)
  plsc.subcore_barrier()

  pltpu.sync_copy(dst_hbm.at[pl.ds(s_id * ng2, ng2)], cb0)
  pltpu.sync_copy(et_hbm.at[pl.ds(s_id * ng2, ng2)], cb1)
  def _comp(i, _):
    j = i // (B // L)
    k = i % (B // L)
    dv = cb0[j, pl.ds(k * L, L)]
    ev = cb1[j, pl.ds(k * L, L)]
    comp_all[j, pl.ds(k * L, L)] = dv * n_rel + ev
    return 0
  lax.fori_loop(0, ng2 * (B // L), _comp, 0)
  def _fire(g, _):
    pltpu.async_copy(ones_v, cnt_sh.at[comp_all.at[g]], sem_a, add=True)
    return 0
  lax.fori_loop(0, ng2, _fire, 0)
  def _drain(g, _):
    pltpu.make_async_copy(ones_v, cnt_sh.at[comp_all.at[g]], sem_a).wait()
    return 0
  lax.fori_loop(0, ng2, _drain, 0)
  plsc.subcore_barrier()

  pltpu.sync_copy(cnt_sh.at[pl.ds(s_id * bins_per_tile, bins_per_tile)], zbuf)
  def _r(i, _):
    v = zbuf[pl.ds(i * L, L)]
    zbuf[pl.ds(i * L, L)] = 1.0 / jnp.maximum(v, 1.0)
    return 0
  lax.fori_loop(0, bins_per_tile // L, _r, 0)
  pltpu.sync_copy(zbuf, cnt_sh.at[pl.ds(s_id * bins_per_tile, bins_per_tile)])
  plsc.subcore_barrier()

  pltpu.sync_copy(src_hbm.at[pl.ds(wid * ng, ng)], cb0.at[pl.ds(0, ng)])
  pltpu.sync_copy(dst_hbm.at[pl.ds(wid * ng, ng)], cb1.at[pl.ds(0, ng)])
  pltpu.sync_copy(et_hbm.at[pl.ds(wid * ng, ng)], etb)
  def _gix(i, _):
    j = i // (B // L)
    k = i % (B // L)
    sv = cb0[j, pl.ds(k * L, L)]
    dv = cb1[j, pl.ds(k * L, L)]
    ev = etb[j, pl.ds(k * L, L)]
    comp_all[j, pl.ds(k * L, L)] = dv * n_rel + ev
    gidx_all[j, pl.ds(k * L, L)] = ev * n_nodes + sv
    return 0
  lax.fori_loop(0, ng * (B // L), _gix, 0)
  def _wfire(g, _):
    pltpu.async_copy(cnt_sh.at[comp_all.at[g]], w_all.at[g], sem_b)
    return 0
  lax.fori_loop(0, ng, _wfire, 0)
  def _wdrain(g, _):
    pltpu.make_async_copy(cnt_sh.at[comp_all.at[g]], w_all.at[g], sem_b).wait()
    return 0
  lax.fori_loop(0, ng, _wdrain, 0)
  pltpu.sync_copy(w_all, w_hbm.at[pl.ds(wid * ng, ng)])
  pltpu.sync_copy(gidx_all, gidx_hbm.at[pl.ds(wid * ng, ng)])


def _make_sc_setup(n_nodes, n_rel, ng, nbins):
  mesh = plsc.VectorSubcoreMesh(core_axis_name="c", subcore_axis_name="s")
  body = functools.partial(_sc_setup_body, n_nodes, n_rel, ng, nbins)
  return pl.kernel(
      body,
      out_type=(
          jax.ShapeDtypeStruct((NW * ng, B), jnp.float32),
          jax.ShapeDtypeStruct((NW * ng, B), jnp.int32),
      ),
      mesh=mesh,
      scratch_types=[
          pltpu.VMEM_SHARED((nbins,), jnp.float32),
          pltpu.VMEM((nbins // NS,), jnp.float32),
          pltpu.VMEM((2 * ng, B), jnp.int32),
          pltpu.VMEM((2 * ng, B), jnp.int32),
          pltpu.VMEM((ng, B), jnp.int32),
          pltpu.VMEM((2 * ng, B), jnp.int32),
          pltpu.VMEM((ng, B), jnp.float32),
          pltpu.VMEM((ng, B), jnp.int32),
          pltpu.VMEM((B,), jnp.float32),
          pltpu.SemaphoreType.DMA,
          pltpu.SemaphoreType.DMA,
      ],
      compiler_params=pltpu.CompilerParams(needs_layout_passes=False),
      name="rgcn_sc_setup",
  )


def _sc_layer_body(n_acc, d, m0, m1, xr_hbm, gidx_hbm, dst_hbm, w_hbm,
                   out_hbm, acc_sh, r0, r1, g0, g1, d0, d1, w0, w1,
                   sg0, sg1, ss0, ss1, se0, se1):
  s_id = lax.axis_index("s")
  c_id = lax.axis_index("c")
  rows = (r0, r1)
  gts = (g0, g1)
  dts = (d0, d1)
  wts = (w0, w1)
  sgs = (sg0, sg1)
  sss = (ss0, ss1)
  ses = (se0, se1)
  m = jnp.where(c_id == 0, m0, m1)
  gbase = jnp.where(c_id == 0, s_id * m0, NS * m0 + s_id * m1)
  nchunks = m // CHUNK

  rows_per_tile = n_acc // NS
  def _z(i, _):
    for c8 in range(d // L):
      r0[i, pl.ds(c8 * L, L)] = jnp.zeros((L,), jnp.float32)
    return 0
  lax.fori_loop(0, B, _z, 0)
  for k in range(rows_per_tile // B):
    pltpu.sync_copy(r0, acc_sh.at[pl.ds(s_id * rows_per_tile + k * B, B)])

  def _ech_start(cc, p):
    off = pl.multiple_of(gbase + cc * CHUNK, 8)
    pltpu.async_copy(gidx_hbm.at[pl.ds(off, CHUNK)], gts[p], ses[p])
    pltpu.async_copy(dst_hbm.at[pl.ds(off, CHUNK)], dts[p], ses[p])
    pltpu.async_copy(w_hbm.at[pl.ds(off, CHUNK)], wts[p], ses[p])

  def _ech_wait(p):
    base = pl.multiple_of(gbase, 8)
    pltpu.make_async_copy(gidx_hbm.at[pl.ds(base, CHUNK)], gts[p],
                          ses[p]).wait()
    pltpu.make_async_copy(dst_hbm.at[pl.ds(base, CHUNK)], dts[p],
                          ses[p]).wait()
    pltpu.make_async_copy(w_hbm.at[pl.ds(base, CHUNK)], wts[p],
                          ses[p]).wait()

  _ech_start(0, 0)
  plsc.subcore_barrier()

  def _scale(buf, wt, gg):
    def _body(j, _):
      wv = wt[gg, pl.ds(j * L, L)]
      for k in range(L):
        w = wv[k]
        i = j * L + k
        for c8 in range(d // L):
          buf[i, pl.ds(c8 * L, L)] = buf[i, pl.ds(c8 * L, L)] * w
      return 0
    lax.fori_loop(0, B // L, _body, 0)

  def _gwait(b):
    pltpu.make_async_copy(xr_hbm.at[gts[0].at[0]], rows[b], sgs[b]).wait()

  def _swait(b):
    pltpu.make_async_copy(rows[b], acc_sh.at[dts[0].at[0]], sss[b]).wait()

  def _chunk(cc, p):
    _ech_wait(p)
    @pl.when(cc + 1 < nchunks)
    def _pref():
      _ech_start(cc + 1, 1 - p)
    for pair in range(CHUNK // 2):
      gg0 = 2 * pair
      gg1 = 2 * pair + 1
      glob0 = cc * CHUNK + gg0
      @pl.when(glob0 >= 2)
      def _w0():
        _swait(0)
        _swait(1)
      pltpu.async_copy(xr_hbm.at[gts[p].at[gg0]], rows[0], sgs[0])
      pltpu.async_copy(xr_hbm.at[gts[p].at[gg1]], rows[1], sgs[1])
      _gwait(0)
      _scale(r0, wts[p], gg0)
      pltpu.async_copy(r0, acc_sh.at[dts[p].at[gg0]], sss[0], add=True)
      _gwait(1)
      _scale(r1, wts[p], gg1)
      pltpu.async_copy(r1, acc_sh.at[dts[p].at[gg1]], sss[1], add=True)

  def _iter(t, _):
    _chunk(2 * t, 0)
    _chunk(2 * t + 1, 1)
    return 0
  lax.fori_loop(0, nchunks // 2, _iter, 0)
  _swait(0)
  _swait(1)
  plsc.subcore_barrier()

  base = pl.multiple_of(c_id * n_acc + s_id * rows_per_tile, 8)
  pltpu.sync_copy(acc_sh.at[pl.ds(s_id * rows_per_tile, rows_per_tile)],
                  out_hbm.at[pl.ds(base, rows_per_tile)])


def _make_sc_layer(n_acc, d, m0, m1):
  mesh = plsc.VectorSubcoreMesh(core_axis_name="c", subcore_axis_name="s")
  body = functools.partial(_sc_layer_body, n_acc, d, m0, m1)
  return pl.kernel(
      body,
      out_type=jax.ShapeDtypeStruct((NC * n_acc, d), jnp.float32),
      mesh=mesh,
      scratch_types=[
          pltpu.VMEM_SHARED((n_acc, d), jnp.float32),
          pltpu.VMEM((B, d), jnp.float32),
          pltpu.VMEM((B, d), jnp.float32),
          pltpu.VMEM((CHUNK, B), jnp.int32),
          pltpu.VMEM((CHUNK, B), jnp.int32),
          pltpu.VMEM((CHUNK, B), jnp.int32),
          pltpu.VMEM((CHUNK, B), jnp.int32),
          pltpu.VMEM((CHUNK, B), jnp.float32),
          pltpu.VMEM((CHUNK, B), jnp.float32),
          pltpu.SemaphoreType.DMA,
          pltpu.SemaphoreType.DMA,
          pltpu.SemaphoreType.DMA,
          pltpu.SemaphoreType.DMA,
          pltpu.SemaphoreType.DMA,
          pltpu.SemaphoreType.DMA,
      ],
      name="rgcn_sc_layer",
  )


def _mm_body(x_ref, w_ref, o_ref):
  o_ref[0] = jnp.dot(x_ref[...], w_ref[0],
                     preferred_element_type=jnp.float32)


def _relation_matmul(x, w_stack, n_blk):
  n, d = x.shape
  r1 = w_stack.shape[0]
  grid = (r1, n // n_blk)
  return pl.pallas_call(
      _mm_body,
      grid=grid,
      in_specs=[
          pl.BlockSpec((n_blk, d), lambda r, i: (i, 0)),
          pl.BlockSpec((1, d, d), lambda r, i: (r, 0, 0)),
      ],
      out_specs=pl.BlockSpec((1, n_blk, d), lambda r, i: (r, i, 0)),
      out_shape=jax.ShapeDtypeStruct((r1, n, d), jnp.float32),
  )(x, w_stack)


def _combine_body(acc_ref0, acc_ref1, xr_ref, b_ref, o_ref):
  o_ref[...] = jnp.maximum(
      acc_ref0[...] + acc_ref1[...] + xr_ref[0] + b_ref[...], 0.0)


def _combine(acc, xr, r, b, n_blk):
  n_acc2, d = acc.shape
  n_acc = n_acc2 // 2
  grid = (n_acc // n_blk,)
  nb = n_acc // n_blk
  return pl.pallas_call(
      _combine_body,
      grid=grid,
      in_specs=[
          pl.BlockSpec((n_blk, d), lambda i: (i, 0)),
          pl.BlockSpec((n_blk, d), lambda i: (i + nb, 0)),
          pl.BlockSpec((1, n_blk, d), lambda i: (r, i, 0)),
          pl.BlockSpec((1, d), lambda i: (0, 0)),
      ],
      out_specs=pl.BlockSpec((n_blk, d), lambda i: (i, 0)),
      out_shape=jax.ShapeDtypeStruct((n_acc, d), jnp.float32),
  )(acc, acc, xr, b.reshape(1, d))


def _pool_body(n_groups, h_ref, batch_ref, wc_ref, bc_ref, o_ref):
  npad = h_ref.shape[0]
  ids = lax.broadcasted_iota(jnp.int32, (n_groups, npad), 0)
  onehot = jnp.where(ids == batch_ref[...], 1.0, 0.0)
  sums = jnp.dot(onehot, h_ref[...], preferred_element_type=jnp.float32)
  cnt = jnp.sum(onehot, axis=1, keepdims=True)
  g = sums / jnp.maximum(cnt, 1.0)
  o_ref[...] = jnp.dot(g, wc_ref[...],
                       preferred_element_type=jnp.float32) + bc_ref[...]


def _pool_classify(h_pad, batch_pad, wc, bc, n_groups):
  npad, d = h_pad.shape
  c = wc.shape[1]
  return pl.pallas_call(
      functools.partial(_pool_body, n_groups),
      in_specs=[
          pl.BlockSpec((npad, d), lambda: (0, 0)),
          pl.BlockSpec((1, npad), lambda: (0, 0)),
          pl.BlockSpec((d, c), lambda: (0, 0)),
          pl.BlockSpec((1, c), lambda: (0, 0)),
      ],
      out_specs=pl.BlockSpec((n_groups, c), lambda: (0, 0)),
      out_shape=jax.ShapeDtypeStruct((n_groups, c), jnp.float32),
  )(h_pad, batch_pad.reshape(1, npad), wc, bc.reshape(1, c))


def kernel(x, edge_index, edge_type, batch, W1, root1, b1, W2, root2, b2,
           Wc, bc):
  n, d = x.shape
  r = W1.shape[0]
  e = edge_index.shape[1]
  n_groups = NUM_GRAPHS

  src = edge_index[0].astype(jnp.int32)
  dst = edge_index[1].astype(jnp.int32)
  et = edge_type.astype(jnp.int32)
  batch32 = batch.astype(jnp.int32)

  ng = _ceil_to(-(-e // (NW * B)), 2 * CHUNK)
  e_pad = NW * B * ng
  n_acc = _ceil_to(n + 1, NS * B)
  nbins = n_acc * r
  pad = e_pad - e

  dummy_dst = n + jnp.arange(pad, dtype=jnp.int32) % (n_acc - n)
  src_p = jnp.concatenate([src, jnp.zeros((pad,), jnp.int32)]).reshape(-1, B)
  dst_p = jnp.concatenate([dst, dummy_dst]).reshape(-1, B)
  et_p = jnp.concatenate([et, jnp.zeros((pad,), jnp.int32)]).reshape(-1, B)

  w_edge, gidx = _make_sc_setup(n_acc, r, ng, nbins)(dst_p, et_p, src_p)

  m0, m1 = 144, 16
  sc_layer = _make_sc_layer(n_acc, d, m0, m1)
  n_blk = 1024

  x_p = jnp.pad(x, ((0, n_acc - n), (0, 0)))

  def layer(h, w_rel, root, b):
    w_stack = jnp.concatenate([w_rel, root[None]], axis=0)
    xr = _relation_matmul(h, w_stack, n_blk)
    xr_flat = xr[:r].reshape(r * n_acc, d)
    acc = sc_layer(xr_flat, gidx, dst_p, w_edge)
    return _combine(acc, xr, r, b, n_blk)

  h = layer(x_p, W1, root1, b1)
  h = layer(h, W2, root2, b2)

  batch_pad = jnp.concatenate(
      [batch32, jnp.full((n_acc - n,), n_groups, jnp.int32)])
  return _pool_classify(h, batch_pad, Wc, bc, n_groups)
